# baseline (speedup 1.0000x reference)
import functools

import jax
import jax.numpy as jnp
from jax import lax
from jax.experimental import pallas as pl
from jax.experimental.pallas import tpu as pltpu
from jax.experimental.pallas import tpu_sc as plsc

D_MODEL = 4096
NUM_CORES = 2
NUM_SUBCORES = 16
NUM_WORKERS = NUM_CORES * NUM_SUBCORES
CHUNK = 8
NBUF = 3


def _emb_body(n_chunks, b_per_w, idx_hbm, table_hbm, out_hbm, idx_v, rows_v, gsem, osem):
    wid = lax.axis_index("s") * NUM_CORES + lax.axis_index("c")
    base = wid * b_per_w

    def gather(j, b):
        pltpu.async_copy(table_hbm.at[idx_v.at[j]], rows_v.at[b], gsem.at[b])

    def gwait(b):
        pltpu.make_async_copy(
            table_hbm.at[idx_v.at[0]], rows_v.at[b], gsem.at[b]
        ).wait()

    def scatter(j, b):
        pltpu.async_copy(
            rows_v.at[b], out_hbm.at[pl.ds(base + j * CHUNK, CHUNK)], osem.at[b]
        )

    def owait(b):
        pltpu.make_async_copy(
            rows_v.at[b], out_hbm.at[pl.ds(0, CHUNK)], osem.at[b]
        ).wait()

    pltpu.sync_copy(idx_hbm.at[wid], idx_v)

    gather(0, 0)
    gather(1, 1)
    gwait(0)
    scatter(0, 0)
    gather(2, 2)

    @pl.loop(0, (n_chunks - 5) // 3)
    def _main(g):
        j0 = 3 * g + 1
        for i in range(3):
            j = j0 + i
            b = (1 + i) % 3
            bp = i
            gwait(b)
            scatter(j, b)
            owait(bp)
            gather(j + 2, bp)

    for j, nxt in ((n_chunks - 4, n_chunks - 2),
                   (n_chunks - 3, n_chunks - 1),
                   (n_chunks - 2, None),
                   (n_chunks - 1, None)):
        b = j % 3
        bp = (j - 1) % 3
        gwait(b)
        scatter(j, b)
        owait(bp)
        if nxt is not None:
            gather(nxt, bp)
    owait((n_chunks - 1) % 3)


@functools.partial(jax.jit, static_argnames=("n_tokens",))
def _embed(idx_grouped, embed_table, n_tokens):
    b_per_w = n_tokens // NUM_WORKERS
    n_chunks = b_per_w // CHUNK
    assert (n_chunks - 5) % 3 == 0 and n_chunks >= 8
    mesh = plsc.VectorSubcoreMesh(
        core_axis_name="c",
        subcore_axis_name="s",
        num_cores=NUM_CORES,
        num_subcores=NUM_SUBCORES,
    )
    run = pl.kernel(
        functools.partial(_emb_body, n_chunks, b_per_w),
        out_type=jax.ShapeDtypeStruct((n_tokens, D_MODEL), jnp.float32),
        mesh=mesh,
        scratch_types=[
            pltpu.VMEM((n_chunks, CHUNK), jnp.int32),
            pltpu.VMEM((NBUF, CHUNK, D_MODEL), jnp.float32),
            pltpu.SemaphoreType.DMA((NBUF,)),
            pltpu.SemaphoreType.DMA((NBUF,)),
        ],
    )
    return run(idx_grouped, embed_table)


def kernel(input_ids, embed_table):
    batch, seq = input_ids.shape
    n_tokens = batch * seq
    b_per_w = n_tokens // NUM_WORKERS
    n_chunks = b_per_w // CHUNK
    idx_grouped = input_ids.reshape(NUM_WORKERS, n_chunks, CHUNK).astype(jnp.int32)
    out = _embed(idx_grouped, embed_table, n_tokens)
    return out.reshape(batch, seq, embed_table.shape[1])


# speedup vs baseline: 1.0116x; 1.0116x over previous
import functools

import jax
import jax.numpy as jnp
from jax import lax
from jax.experimental import pallas as pl
from jax.experimental.pallas import tpu as pltpu
from jax.experimental.pallas import tpu_sc as plsc

D_MODEL = 4096
NUM_CORES = 2
NUM_SUBCORES = 16
NUM_WORKERS = NUM_CORES * NUM_SUBCORES
CHUNK = 8
NBUF = 2


def _emb_body(n_chunks, b_per_w, idx_hbm, table_hbm, out_hbm, idx_v, rows_v, gsem):
    wid = lax.axis_index("s") * NUM_CORES + lax.axis_index("c")
    base = wid * b_per_w
    seq = idx_hbm.shape[1]
    per_row = seq // b_per_w

    def gather(j, b):
        pltpu.async_copy(
            table_hbm.at[idx_v.at[pl.ds(j * CHUNK, CHUNK)]], rows_v.at[b], gsem.at[b]
        )

    def gwait(b):
        pltpu.make_async_copy(
            table_hbm.at[idx_v.at[pl.ds(0, CHUNK)]], rows_v.at[b], gsem.at[b]
        ).wait()

    def writeback(j, b):
        pltpu.sync_copy(rows_v.at[b], out_hbm.at[pl.ds(base + j * CHUNK, CHUNK)])

    pltpu.sync_copy(
        idx_hbm.at[wid // per_row, pl.ds((wid % per_row) * b_per_w, b_per_w)],
        idx_v,
    )

    gather(0, 0)
    gather(1, 1)

    @pl.loop(0, (n_chunks - NBUF) // NBUF)
    def _main(g):
        j0 = NBUF * g
        for b in range(NBUF):
            gwait(b)
            writeback(j0 + b, b)
            gather(j0 + b + NBUF, b)

    for b in range(NBUF):
        gwait(b)
        writeback(n_chunks - NBUF + b, b)


@functools.partial(jax.jit, static_argnames=("n_tokens",))
def _embed(input_ids, embed_table, n_tokens):
    b_per_w = n_tokens // NUM_WORKERS
    n_chunks = b_per_w // CHUNK
    assert n_chunks % NBUF == 0 and b_per_w % CHUNK == 0
    mesh = plsc.VectorSubcoreMesh(
        core_axis_name="c",
        subcore_axis_name="s",
        num_cores=NUM_CORES,
        num_subcores=NUM_SUBCORES,
    )
    run = pl.kernel(
        functools.partial(_emb_body, n_chunks, b_per_w),
        out_type=jax.ShapeDtypeStruct((n_tokens, D_MODEL), jnp.float32),
        mesh=mesh,
        scratch_types=[
            pltpu.VMEM((b_per_w,), jnp.int32),
            pltpu.VMEM((NBUF, CHUNK, D_MODEL), jnp.float32),
            pltpu.SemaphoreType.DMA((NBUF,)),
        ],
    )
    return run(input_ids, embed_table)


def kernel(input_ids, embed_table):
    batch, seq = input_ids.shape
    out = _embed(input_ids.astype(jnp.int32), embed_table, batch * seq)
    return out.reshape(batch, seq, embed_table.shape[1])
